# Initial kernel scaffold; baseline (speedup 1.0000x reference)
import jax
import jax.numpy as jnp
from jax.experimental import pallas as pl


def kernel(x, W_enc, b_enc, W_dec, b_dec):
    raise NotImplementedError("write your pallas kernel here")


# speedup vs baseline: 23.3994x; 23.3994x over previous
import functools

import jax
import jax.numpy as jnp
from jax import lax
from jax.experimental import pallas as pl
from jax.experimental.pallas import tpu as pltpu
from jax.experimental.pallas import tpu_sc as plsc

_B = 2048
_D = 2048
_NL = 8192
_K_TOTAL = 32 * _B
_TOTAL = _B * _NL

_NW = 32
_NC = 2
_PER_W = _TOTAL // _NW
_CH = 16384
_NB1 = 1 << 16
_NB2 = 1 << 15


def _enc_body(x_ref, w_ref, b_ref, o_ref):
    acc = lax.dot_general(
        x_ref[...], w_ref[...], (((1,), (1,)), ((), ())),
        preferred_element_type=jnp.float32)
    o_ref[...] = jnp.maximum(acc + b_ref[...], 0.0)


def _encode(x, W_enc, b_enc):
    jblk = 1024
    return pl.pallas_call(
        _enc_body,
        grid=(_NL // jblk,),
        in_specs=[
            pl.BlockSpec((_B, _D), lambda j: (0, 0)),
            pl.BlockSpec((jblk, _D), lambda j: (j, 0)),
            pl.BlockSpec((1, jblk), lambda j: (0, j)),
        ],
        out_specs=pl.BlockSpec((_B, jblk), lambda j: (0, j)),
        out_shape=jax.ShapeDtypeStruct((_B, _NL), jnp.float32),
    )(x, W_enc, b_enc.reshape(1, _NL))


_SC_MESH = plsc.VectorSubcoreMesh(core_axis_name="c", subcore_axis_name="s")
_SC_PARAMS = pltpu.CompilerParams(needs_layout_passes=False)


@functools.partial(
    pl.kernel,
    out_type=jax.ShapeDtypeStruct((_NW, _NB1), jnp.int32),
    mesh=_SC_MESH,
    compiler_params=_SC_PARAMS,
    scratch_types=[
        pltpu.VMEM((_NB1,), jnp.int32),
        pltpu.VMEM((_CH,), jnp.float32),
    ],
)
def _hist1(pre_hbm, out_hbm, hist_v, chunk_v):
    wid = lax.axis_index("s") * _NC + lax.axis_index("c")

    def zbody(i, _):
        hist_v[pl.ds(i * 16, 16)] = jnp.zeros((16,), jnp.int32)
        return 0
    lax.fori_loop(0, _NB1 // 16, zbody, 0)

    base = wid * _PER_W
    ones = jnp.ones((16,), jnp.int32)

    def cbody(c, _):
        pltpu.sync_copy(pre_hbm.at[pl.ds(base + c * _CH, _CH)], chunk_v)

        def ibody(i, _):
            v = chunk_v[pl.ds(i * 16, 16)]
            bits = plsc.bitcast(v, jnp.int32)
            bkt = bits >> 15
            plsc.addupdate_scatter(hist_v, [bkt], ones)
            return 0
        lax.fori_loop(0, _CH // 16, ibody, 0)
        return 0
    lax.fori_loop(0, _PER_W // _CH, cbody, 0)

    pltpu.sync_copy(hist_v, out_hbm.at[wid])


@functools.partial(
    pl.kernel,
    out_type=jax.ShapeDtypeStruct((_NW, _NB2), jnp.int32),
    mesh=_SC_MESH,
    compiler_params=_SC_PARAMS,
    scratch_types=[
        pltpu.VMEM((_NB2,), jnp.int32),
        pltpu.VMEM((_CH,), jnp.float32),
        pltpu.VMEM((16,), jnp.int32),
    ],
)
def _hist2(pre_hbm, b1_hbm, out_hbm, hist_v, chunk_v, bvec_v):
    wid = lax.axis_index("s") * _NC + lax.axis_index("c")

    def zbody(i, _):
        hist_v[pl.ds(i * 16, 16)] = jnp.zeros((16,), jnp.int32)
        return 0
    lax.fori_loop(0, _NB2 // 16, zbody, 0)

    pltpu.sync_copy(b1_hbm.at[pl.ds(0, 16)], bvec_v)
    b1 = bvec_v[pl.ds(0, 16)][0]

    base = wid * _PER_W
    ones = jnp.ones((16,), jnp.int32)

    def cbody(c, _):
        pltpu.sync_copy(pre_hbm.at[pl.ds(base + c * _CH, _CH)], chunk_v)

        def ibody(i, _):
            v = chunk_v[pl.ds(i * 16, 16)]
            bits = plsc.bitcast(v, jnp.int32)
            sel = (bits >> 15) == b1
            bkt = bits & 0x7FFF
            plsc.addupdate_scatter(hist_v, [bkt], ones, mask=sel)
            return 0
        lax.fori_loop(0, _CH // 16, ibody, 0)
        return 0
    lax.fori_loop(0, _PER_W // _CH, cbody, 0)

    pltpu.sync_copy(hist_v, out_hbm.at[wid])


def _finder_body(nr, kt_ref, h_ref, b_ref, r_ref):
    kf = kt_ref[0, 0].astype(jnp.float32)
    hh = h_ref[...].astype(jnp.float32)
    h2 = jnp.sum(hh, axis=0)

    ic = lax.broadcasted_iota(jnp.int32, (128, 128), 0)
    jc = lax.broadcasted_iota(jnp.int32, (128, 128), 1)
    tc = (ic >= jc).astype(jnp.float32)
    s_in = lax.dot_general(h2, tc, (((1,), (0,)), ((), ())),
                           preferred_element_type=jnp.float32,
                           precision=lax.Precision.HIGHEST)
    t = s_in[:, 0:1]
    ir = lax.broadcasted_iota(jnp.int32, (nr, nr), 0)
    jr = lax.broadcasted_iota(jnp.int32, (nr, nr), 1)
    ar = (jr > ir).astype(jnp.float32)
    rows_after = lax.dot_general(ar, t, (((1,), (0,)), ((), ())),
                                 preferred_element_type=jnp.float32,
                                 precision=lax.Precision.HIGHEST)
    s = s_in + rows_after

    cnt = jnp.sum((s >= kf).astype(jnp.float32))
    bstar = cnt.astype(jnp.int32) - 1
    s_next = jnp.max(jnp.where(s < kf, s, 0.0))
    r = (kf - s_next).astype(jnp.int32)
    b_ref[...] = jnp.full((8, 128), bstar, jnp.int32)
    r_ref[...] = jnp.full((8, 128), r, jnp.int32)


def _find_boundary(hists, ktarget, nb):
    nr = nb // 128
    return pl.pallas_call(
        functools.partial(_finder_body, nr),
        in_specs=[
            pl.BlockSpec(memory_space=pltpu.SMEM),
            pl.BlockSpec((_NW, nr, 128), lambda: (0, 0, 0)),
        ],
        out_specs=[
            pl.BlockSpec((8, 128), lambda: (0, 0)),
            pl.BlockSpec((8, 128), lambda: (0, 0)),
        ],
        out_shape=[
            jax.ShapeDtypeStruct((8, 128), jnp.int32),
            jax.ShapeDtypeStruct((8, 128), jnp.int32),
        ],
    )(ktarget, hists.reshape(_NW, nr, 128))


def _dec_body(thr_ref, pre_ref, w_ref, bd_ref, sp_ref, rec_ref):
    t = thr_ref[0, 0]
    p = pre_ref[...]
    sp = jnp.where(p >= t, p, 0.0)
    sp_ref[...] = sp
    contrib = lax.dot_general(
        sp, w_ref[...], (((1,), (1,)), ((), ())),
        preferred_element_type=jnp.float32)
    k = pl.program_id(0)

    @pl.when(k == 0)
    def _():
        rec_ref[...] = bd_ref[...] + contrib

    @pl.when(k > 0)
    def _():
        rec_ref[...] += contrib


def _decode(pre, W_dec, b_dec, thr):
    kblk = 512
    return pl.pallas_call(
        _dec_body,
        grid=(_NL // kblk,),
        in_specs=[
            pl.BlockSpec(memory_space=pltpu.SMEM),
            pl.BlockSpec((_B, kblk), lambda k: (0, k)),
            pl.BlockSpec((_D, kblk), lambda k: (0, k)),
            pl.BlockSpec((1, _D), lambda k: (0, 0)),
        ],
        out_specs=[
            pl.BlockSpec((_B, kblk), lambda k: (0, k)),
            pl.BlockSpec((_B, _D), lambda k: (0, 0)),
        ],
        out_shape=[
            jax.ShapeDtypeStruct((_B, _NL), jnp.float32),
            jax.ShapeDtypeStruct((_B, _D), jnp.float32),
        ],
    )(thr, pre, W_dec, b_dec.reshape(1, _D))


def kernel(x, W_enc, b_enc, W_dec, b_dec):
    pre = _encode(x, W_enc, b_enc)
    flat = pre.reshape(-1)
    h1 = _hist1(flat)
    kt = jnp.full((1, 1), _K_TOTAL, jnp.int32)
    b1_arr, r_arr = _find_boundary(h1, kt, _NB1)
    h2 = _hist2(flat, b1_arr.reshape(-1))
    b2_arr, _ = _find_boundary(h2, r_arr[0:1, 0:1], _NB2)
    tbits = (b1_arr[0, 0] << 15) | b2_arr[0, 0]
    thr = lax.bitcast_convert_type(tbits, jnp.float32).reshape(1, 1)
    sparse_acts, reconstruction = _decode(pre, W_dec, b_dec, thr)
    return (reconstruction, sparse_acts)


# speedup vs baseline: 29.7316x; 1.2706x over previous
import functools

import jax
import jax.numpy as jnp
from jax import lax
from jax.experimental import pallas as pl
from jax.experimental.pallas import tpu as pltpu
from jax.experimental.pallas import tpu_sc as plsc

_B = 2048
_D = 2048
_NL = 8192
_K_TOTAL = 32 * _B
_TOTAL = _B * _NL

_NW = 32
_NC = 2
_PER_W = _TOTAL // _NW
_CH = 16384
_NB1 = 1 << 16
_NB2 = 1 << 15


def _enc_body(x_ref, w_ref, b_ref, o_ref):
    acc = lax.dot_general(
        x_ref[...], w_ref[...], (((1,), (1,)), ((), ())),
        preferred_element_type=jnp.float32)
    o_ref[...] = jnp.maximum(acc + b_ref[...], 0.0)


def _encode(x, W_enc, b_enc):
    jblk = 1024
    return pl.pallas_call(
        _enc_body,
        grid=(_NL // jblk,),
        in_specs=[
            pl.BlockSpec((_B, _D), lambda j: (0, 0)),
            pl.BlockSpec((jblk, _D), lambda j: (j, 0)),
            pl.BlockSpec((1, jblk), lambda j: (0, j)),
        ],
        out_specs=pl.BlockSpec((_B, jblk), lambda j: (0, j)),
        out_shape=jax.ShapeDtypeStruct((_B, _NL), jnp.float32),
    )(x, W_enc, b_enc.reshape(1, _NL))


_SC_MESH = plsc.VectorSubcoreMesh(core_axis_name="c", subcore_axis_name="s")
_SC_PARAMS = pltpu.CompilerParams(needs_layout_passes=False)


_UN = 16
_NCHUNK = _PER_W // _CH


def _zero_hist(hist_v, nb):
    def zbody(i, _):
        off = i * 256
        for u in range(16):
            hist_v[pl.ds(off + u * 16, 16)] = jnp.zeros((16,), jnp.int32)
        return 0
    lax.fori_loop(0, nb // 256, zbody, 0)


@functools.partial(
    pl.kernel,
    out_type=jax.ShapeDtypeStruct((_NW, _NB1), jnp.int32),
    mesh=_SC_MESH,
    compiler_params=_SC_PARAMS,
    scratch_types=[
        pltpu.VMEM((_NB1,), jnp.int32),
        pltpu.VMEM((_CH,), jnp.float32),
        pltpu.VMEM((_CH,), jnp.float32),
        pltpu.SemaphoreType.DMA,
        pltpu.SemaphoreType.DMA,
    ],
)
def _hist1(pre_hbm, out_hbm, hist_v, chunk0_v, chunk1_v, sem0, sem1):
    wid = lax.axis_index("s") * _NC + lax.axis_index("c")
    base = wid * _PER_W
    ones = jnp.ones((16,), jnp.int32)

    bufs = (chunk0_v, chunk1_v)
    for b in range(2):
        pltpu.async_copy(pre_hbm.at[pl.ds(base + b * _CH, _CH)],
                         bufs[b], (sem0, sem1)[b])
    _zero_hist(hist_v, _NB1)

    def cbody(k, _):
        for b in range(2):
            c = k * 2 + b
            sem = (sem0, sem1)[b]
            pltpu.make_async_copy(pre_hbm.at[pl.ds(base + c * _CH, _CH)],
                                  bufs[b], sem).wait()
            cv = bufs[b]

            def ibody(i, _):
                off = i * (16 * _UN)
                for u in range(_UN):
                    v = cv[pl.ds(off + u * 16, 16)]
                    bits = plsc.bitcast(v, jnp.int32)
                    plsc.addupdate_scatter(hist_v, [bits >> 15], ones,
                                           mask=bits != 0)
                return 0
            lax.fori_loop(0, _CH // (16 * _UN), ibody, 0)

            nxt = c + 2

            @pl.when(nxt < _NCHUNK)
            def _():
                pltpu.async_copy(pre_hbm.at[pl.ds(base + nxt * _CH, _CH)],
                                 bufs[b], sem)
        return 0
    lax.fori_loop(0, _NCHUNK // 2, cbody, 0)

    pltpu.sync_copy(hist_v, out_hbm.at[wid])


@functools.partial(
    pl.kernel,
    out_type=jax.ShapeDtypeStruct((_NW, _NB2), jnp.int32),
    mesh=_SC_MESH,
    compiler_params=_SC_PARAMS,
    scratch_types=[
        pltpu.VMEM((_NB2,), jnp.int32),
        pltpu.VMEM((_CH,), jnp.float32),
        pltpu.VMEM((_CH,), jnp.float32),
        pltpu.VMEM((16,), jnp.int32),
        pltpu.SemaphoreType.DMA,
        pltpu.SemaphoreType.DMA,
    ],
)
def _hist2(pre_hbm, b1_hbm, out_hbm, hist_v, chunk0_v, chunk1_v, bvec_v, sem0, sem1):
    wid = lax.axis_index("s") * _NC + lax.axis_index("c")
    base = wid * _PER_W
    ones = jnp.ones((16,), jnp.int32)

    bufs = (chunk0_v, chunk1_v)
    for b in range(2):
        pltpu.async_copy(pre_hbm.at[pl.ds(base + b * _CH, _CH)],
                         bufs[b], (sem0, sem1)[b])
    _zero_hist(hist_v, _NB2)
    pltpu.sync_copy(b1_hbm.at[pl.ds(0, 16)], bvec_v)
    b1 = bvec_v[pl.ds(0, 16)][0]

    def cbody(k, _):
        for b in range(2):
            c = k * 2 + b
            sem = (sem0, sem1)[b]
            pltpu.make_async_copy(pre_hbm.at[pl.ds(base + c * _CH, _CH)],
                                  bufs[b], sem).wait()
            cv = bufs[b]

            def ibody(i, _):
                off = i * (16 * _UN)
                for u in range(_UN):
                    v = cv[pl.ds(off + u * 16, 16)]
                    bits = plsc.bitcast(v, jnp.int32)
                    sel = (bits >> 15) == b1
                    plsc.addupdate_scatter(hist_v, [bits & 0x7FFF], ones,
                                           mask=sel)
                return 0
            lax.fori_loop(0, _CH // (16 * _UN), ibody, 0)

            nxt = c + 2

            @pl.when(nxt < _NCHUNK)
            def _():
                pltpu.async_copy(pre_hbm.at[pl.ds(base + nxt * _CH, _CH)],
                                 bufs[b], sem)
        return 0
    lax.fori_loop(0, _NCHUNK // 2, cbody, 0)

    pltpu.sync_copy(hist_v, out_hbm.at[wid])


def _finder_body(nr, kt_ref, h_ref, b_ref, r_ref):
    kf = kt_ref[0, 0].astype(jnp.float32)
    hh = h_ref[...].astype(jnp.float32)
    h2 = jnp.sum(hh, axis=0)

    ic = lax.broadcasted_iota(jnp.int32, (128, 128), 0)
    jc = lax.broadcasted_iota(jnp.int32, (128, 128), 1)
    tc = (ic >= jc).astype(jnp.float32)
    s_in = lax.dot_general(h2, tc, (((1,), (0,)), ((), ())),
                           preferred_element_type=jnp.float32,
                           precision=lax.Precision.HIGHEST)
    t = s_in[:, 0:1]
    ir = lax.broadcasted_iota(jnp.int32, (nr, nr), 0)
    jr = lax.broadcasted_iota(jnp.int32, (nr, nr), 1)
    ar = (jr > ir).astype(jnp.float32)
    rows_after = lax.dot_general(ar, t, (((1,), (0,)), ((), ())),
                                 preferred_element_type=jnp.float32,
                                 precision=lax.Precision.HIGHEST)
    s = s_in + rows_after

    cnt = jnp.sum((s >= kf).astype(jnp.float32))
    bstar = jnp.maximum(cnt.astype(jnp.int32) - 1, 0)
    one_hot = ((lax.broadcasted_iota(jnp.int32, (nr, 128), 0) == 0) &
               (lax.broadcasted_iota(jnp.int32, (nr, 128), 1) == 0))
    h00 = jnp.sum(jnp.where(one_hot, h2, 0.0))
    s00 = jnp.sum(jnp.where(one_hot, s, 0.0))
    s_next = jnp.where(cnt > 0.0,
                       jnp.max(jnp.where(s < kf, s, 0.0)),
                       s00 - h00)
    r = (kf - s_next).astype(jnp.int32)
    b_ref[...] = jnp.full((8, 128), bstar, jnp.int32)
    r_ref[...] = jnp.full((8, 128), r, jnp.int32)


def _find_boundary(hists, ktarget, nb):
    nr = nb // 128
    return pl.pallas_call(
        functools.partial(_finder_body, nr),
        in_specs=[
            pl.BlockSpec(memory_space=pltpu.SMEM),
            pl.BlockSpec((_NW, nr, 128), lambda: (0, 0, 0)),
        ],
        out_specs=[
            pl.BlockSpec((8, 128), lambda: (0, 0)),
            pl.BlockSpec((8, 128), lambda: (0, 0)),
        ],
        out_shape=[
            jax.ShapeDtypeStruct((8, 128), jnp.int32),
            jax.ShapeDtypeStruct((8, 128), jnp.int32),
        ],
    )(ktarget, hists.reshape(_NW, nr, 128))


def _dec_body(thr_ref, pre_ref, w_ref, bd_ref, sp_ref, rec_ref):
    t = thr_ref[0, 0]
    p = pre_ref[...]
    sp = jnp.where(p >= t, p, 0.0)
    sp_ref[...] = sp
    contrib = lax.dot_general(
        sp, w_ref[...], (((1,), (1,)), ((), ())),
        preferred_element_type=jnp.float32)
    k = pl.program_id(0)

    @pl.when(k == 0)
    def _():
        rec_ref[...] = bd_ref[...] + contrib

    @pl.when(k > 0)
    def _():
        rec_ref[...] += contrib


def _decode(pre, W_dec, b_dec, thr):
    kblk = 512
    return pl.pallas_call(
        _dec_body,
        grid=(_NL // kblk,),
        in_specs=[
            pl.BlockSpec(memory_space=pltpu.SMEM),
            pl.BlockSpec((_B, kblk), lambda k: (0, k)),
            pl.BlockSpec((_D, kblk), lambda k: (0, k)),
            pl.BlockSpec((1, _D), lambda k: (0, 0)),
        ],
        out_specs=[
            pl.BlockSpec((_B, kblk), lambda k: (0, k)),
            pl.BlockSpec((_B, _D), lambda k: (0, 0)),
        ],
        out_shape=[
            jax.ShapeDtypeStruct((_B, _NL), jnp.float32),
            jax.ShapeDtypeStruct((_B, _D), jnp.float32),
        ],
    )(thr, pre, W_dec, b_dec.reshape(1, _D))


def kernel(x, W_enc, b_enc, W_dec, b_dec):
    pre = _encode(x, W_enc, b_enc)
    flat = pre.reshape(-1)
    h1 = _hist1(flat)
    kt = jnp.full((1, 1), _K_TOTAL, jnp.int32)
    b1_arr, r_arr = _find_boundary(h1, kt, _NB1)
    h2 = _hist2(flat, b1_arr.reshape(-1))
    b2_arr, _ = _find_boundary(h2, r_arr[0:1, 0:1], _NB2)
    tbits = (b1_arr[0, 0] << 15) | b2_arr[0, 0]
    thr = lax.bitcast_convert_type(tbits, jnp.float32).reshape(1, 1)
    sparse_acts, reconstruction = _decode(pre, W_dec, b_dec, thr)
    return (reconstruction, sparse_acts)


# speedup vs baseline: 60.7716x; 2.0440x over previous
import functools

import jax
import jax.numpy as jnp
from jax import lax
from jax.experimental import pallas as pl
from jax.experimental.pallas import tpu as pltpu
from jax.experimental.pallas import tpu_sc as plsc

_B = 2048
_D = 2048
_NL = 8192
_K_TOTAL = 32 * _B
_TOTAL = _B * _NL

_NW = 32
_NC = 2
_PER_W = _TOTAL // _NW
_CH = 16384
_NB1 = 1 << 16
_NB2 = 1 << 15


def _enc_body(x_ref, w_ref, b_ref, o_ref):
    acc = lax.dot_general(
        x_ref[...], w_ref[...], (((1,), (1,)), ((), ())),
        preferred_element_type=jnp.float32)
    o_ref[...] = jnp.maximum(acc + b_ref[...], 0.0)


def _encode(x, W_enc, b_enc):
    jblk = 1024
    return pl.pallas_call(
        _enc_body,
        grid=(_NL // jblk,),
        in_specs=[
            pl.BlockSpec((_B, _D), lambda j: (0, 0)),
            pl.BlockSpec((jblk, _D), lambda j: (j, 0)),
            pl.BlockSpec((1, jblk), lambda j: (0, j)),
        ],
        out_specs=pl.BlockSpec((_B, jblk), lambda j: (0, j)),
        out_shape=jax.ShapeDtypeStruct((_B, _NL), jnp.float32),
    )(x, W_enc, b_enc.reshape(1, _NL))


_SC_MESH = plsc.VectorSubcoreMesh(core_axis_name="c", subcore_axis_name="s")
_SC_PARAMS = pltpu.CompilerParams(needs_layout_passes=False)


_UN = 16
_NCHUNK = _PER_W // _CH


def _zero_hist(hist_v, nb):
    def zbody(i, _):
        off = i * 256
        for u in range(16):
            hist_v[pl.ds(off + u * 16, 16)] = jnp.zeros((16,), jnp.int32)
        return 0
    lax.fori_loop(0, nb // 256, zbody, 0)


@functools.partial(
    pl.kernel,
    out_type=jax.ShapeDtypeStruct((_NW, _NB1), jnp.int32),
    mesh=_SC_MESH,
    compiler_params=_SC_PARAMS,
    scratch_types=[
        pltpu.VMEM((_NB1,), jnp.int32),
        pltpu.VMEM((_CH,), jnp.float32),
        pltpu.VMEM((_CH,), jnp.float32),
        pltpu.SemaphoreType.DMA,
        pltpu.SemaphoreType.DMA,
    ],
)
def _hist1(pre_hbm, out_hbm, hist_v, chunk0_v, chunk1_v, sem0, sem1):
    wid = lax.axis_index("s") * _NC + lax.axis_index("c")
    base = wid * _PER_W
    ones = jnp.ones((16,), jnp.int32)

    bufs = (chunk0_v, chunk1_v)
    for b in range(2):
        pltpu.async_copy(pre_hbm.at[pl.ds(base + b * _CH, _CH)],
                         bufs[b], (sem0, sem1)[b])
    _zero_hist(hist_v, _NB1)

    def cbody(k, _):
        for b in range(2):
            c = k * 2 + b
            sem = (sem0, sem1)[b]
            pltpu.make_async_copy(pre_hbm.at[pl.ds(base + c * _CH, _CH)],
                                  bufs[b], sem).wait()
            cv = bufs[b]

            @plsc.parallel_loop(0, _CH // 16, 1, unroll=_UN)
            def _(i):
                v = cv[pl.ds(i * 16, 16)]
                bits = plsc.bitcast(v, jnp.int32)
                plsc.addupdate_scatter(hist_v, [bits >> 15], ones,
                                       mask=bits != 0)

            nxt = c + 2

            @pl.when(nxt < _NCHUNK)
            def _():
                pltpu.async_copy(pre_hbm.at[pl.ds(base + nxt * _CH, _CH)],
                                 bufs[b], sem)
        return 0
    lax.fori_loop(0, _NCHUNK // 2, cbody, 0)

    pltpu.sync_copy(hist_v, out_hbm.at[wid])


@functools.partial(
    pl.kernel,
    out_type=jax.ShapeDtypeStruct((_NW, _NB2), jnp.int32),
    mesh=_SC_MESH,
    compiler_params=_SC_PARAMS,
    scratch_types=[
        pltpu.VMEM((_NB2,), jnp.int32),
        pltpu.VMEM((_CH,), jnp.float32),
        pltpu.VMEM((_CH,), jnp.float32),
        pltpu.VMEM((16,), jnp.int32),
        pltpu.SemaphoreType.DMA,
        pltpu.SemaphoreType.DMA,
    ],
)
def _hist2(pre_hbm, b1_hbm, out_hbm, hist_v, chunk0_v, chunk1_v, bvec_v, sem0, sem1):
    wid = lax.axis_index("s") * _NC + lax.axis_index("c")
    base = wid * _PER_W
    ones = jnp.ones((16,), jnp.int32)

    bufs = (chunk0_v, chunk1_v)
    for b in range(2):
        pltpu.async_copy(pre_hbm.at[pl.ds(base + b * _CH, _CH)],
                         bufs[b], (sem0, sem1)[b])
    _zero_hist(hist_v, _NB2)
    pltpu.sync_copy(b1_hbm.at[pl.ds(0, 16)], bvec_v)
    b1 = bvec_v[pl.ds(0, 16)][0]

    def cbody(k, _):
        for b in range(2):
            c = k * 2 + b
            sem = (sem0, sem1)[b]
            pltpu.make_async_copy(pre_hbm.at[pl.ds(base + c * _CH, _CH)],
                                  bufs[b], sem).wait()
            cv = bufs[b]

            @plsc.parallel_loop(0, _CH // 16, 1, unroll=_UN)
            def _(i):
                v = cv[pl.ds(i * 16, 16)]
                bits = plsc.bitcast(v, jnp.int32)
                sel = (bits >> 15) == b1
                plsc.addupdate_scatter(hist_v, [bits & 0x7FFF], ones,
                                       mask=sel)

            nxt = c + 2

            @pl.when(nxt < _NCHUNK)
            def _():
                pltpu.async_copy(pre_hbm.at[pl.ds(base + nxt * _CH, _CH)],
                                 bufs[b], sem)
        return 0
    lax.fori_loop(0, _NCHUNK // 2, cbody, 0)

    pltpu.sync_copy(hist_v, out_hbm.at[wid])


def _finder_body(nr, kt_ref, h_ref, b_ref, r_ref):
    kf = kt_ref[0, 0].astype(jnp.float32)
    hh = h_ref[...].astype(jnp.float32)
    h2 = jnp.sum(hh, axis=0)

    ic = lax.broadcasted_iota(jnp.int32, (128, 128), 0)
    jc = lax.broadcasted_iota(jnp.int32, (128, 128), 1)
    tc = (ic >= jc).astype(jnp.float32)
    s_in = lax.dot_general(h2, tc, (((1,), (0,)), ((), ())),
                           preferred_element_type=jnp.float32,
                           precision=lax.Precision.HIGHEST)
    t = s_in[:, 0:1]
    ir = lax.broadcasted_iota(jnp.int32, (nr, nr), 0)
    jr = lax.broadcasted_iota(jnp.int32, (nr, nr), 1)
    ar = (jr > ir).astype(jnp.float32)
    rows_after = lax.dot_general(ar, t, (((1,), (0,)), ((), ())),
                                 preferred_element_type=jnp.float32,
                                 precision=lax.Precision.HIGHEST)
    s = s_in + rows_after

    cnt = jnp.sum((s >= kf).astype(jnp.float32))
    bstar = jnp.maximum(cnt.astype(jnp.int32) - 1, 0)
    one_hot = ((lax.broadcasted_iota(jnp.int32, (nr, 128), 0) == 0) &
               (lax.broadcasted_iota(jnp.int32, (nr, 128), 1) == 0))
    h00 = jnp.sum(jnp.where(one_hot, h2, 0.0))
    s00 = jnp.sum(jnp.where(one_hot, s, 0.0))
    s_next = jnp.where(cnt > 0.0,
                       jnp.max(jnp.where(s < kf, s, 0.0)),
                       s00 - h00)
    r = (kf - s_next).astype(jnp.int32)
    b_ref[...] = jnp.full((8, 128), bstar, jnp.int32)
    r_ref[...] = jnp.full((8, 128), r, jnp.int32)


def _find_boundary(hists, ktarget, nb):
    nr = nb // 128
    return pl.pallas_call(
        functools.partial(_finder_body, nr),
        in_specs=[
            pl.BlockSpec(memory_space=pltpu.SMEM),
            pl.BlockSpec((_NW, nr, 128), lambda: (0, 0, 0)),
        ],
        out_specs=[
            pl.BlockSpec((8, 128), lambda: (0, 0)),
            pl.BlockSpec((8, 128), lambda: (0, 0)),
        ],
        out_shape=[
            jax.ShapeDtypeStruct((8, 128), jnp.int32),
            jax.ShapeDtypeStruct((8, 128), jnp.int32),
        ],
    )(ktarget, hists.reshape(_NW, nr, 128))


def _dec_body(thr_ref, pre_ref, w_ref, bd_ref, sp_ref, rec_ref):
    t = thr_ref[0, 0]
    p = pre_ref[...]
    sp = jnp.where(p >= t, p, 0.0)
    sp_ref[...] = sp
    contrib = lax.dot_general(
        sp, w_ref[...], (((1,), (1,)), ((), ())),
        preferred_element_type=jnp.float32)
    k = pl.program_id(0)

    @pl.when(k == 0)
    def _():
        rec_ref[...] = bd_ref[...] + contrib

    @pl.when(k > 0)
    def _():
        rec_ref[...] += contrib


def _decode(pre, W_dec, b_dec, thr):
    kblk = 512
    return pl.pallas_call(
        _dec_body,
        grid=(_NL // kblk,),
        in_specs=[
            pl.BlockSpec(memory_space=pltpu.SMEM),
            pl.BlockSpec((_B, kblk), lambda k: (0, k)),
            pl.BlockSpec((_D, kblk), lambda k: (0, k)),
            pl.BlockSpec((1, _D), lambda k: (0, 0)),
        ],
        out_specs=[
            pl.BlockSpec((_B, kblk), lambda k: (0, k)),
            pl.BlockSpec((_B, _D), lambda k: (0, 0)),
        ],
        out_shape=[
            jax.ShapeDtypeStruct((_B, _NL), jnp.float32),
            jax.ShapeDtypeStruct((_B, _D), jnp.float32),
        ],
    )(thr, pre, W_dec, b_dec.reshape(1, _D))


def kernel(x, W_enc, b_enc, W_dec, b_dec):
    pre = _encode(x, W_enc, b_enc)
    flat = pre.reshape(-1)
    h1 = _hist1(flat)
    kt = jnp.full((1, 1), _K_TOTAL, jnp.int32)
    b1_arr, r_arr = _find_boundary(h1, kt, _NB1)
    h2 = _hist2(flat, b1_arr.reshape(-1))
    b2_arr, _ = _find_boundary(h2, r_arr[0:1, 0:1], _NB2)
    tbits = (b1_arr[0, 0] << 15) | b2_arr[0, 0]
    thr = lax.bitcast_convert_type(tbits, jnp.float32).reshape(1, 1)
    sparse_acts, reconstruction = _decode(pre, W_dec, b_dec, thr)
    return (reconstruction, sparse_acts)


# speedup vs baseline: 60.7818x; 1.0002x over previous
import functools

import jax
import jax.numpy as jnp
from jax import lax
from jax.experimental import pallas as pl
from jax.experimental.pallas import tpu as pltpu
from jax.experimental.pallas import tpu_sc as plsc

_B = 2048
_D = 2048
_NL = 8192
_K_TOTAL = 32 * _B
_TOTAL = _B * _NL

_NW = 32
_NC = 2
_PER_W = _TOTAL // _NW
_CH1 = 32768
_CH2 = 16384
_NB1 = 1 << 15
_NB2 = 1 << 16


def _enc_body(x_ref, w_ref, b_ref, o_ref):
    acc = lax.dot_general(
        x_ref[...], w_ref[...], (((1,), (1,)), ((), ())),
        preferred_element_type=jnp.float32)
    o_ref[...] = jnp.maximum(acc + b_ref[...], 0.0)


def _encode(x, W_enc, b_enc):
    jblk = 1024
    return pl.pallas_call(
        _enc_body,
        grid=(_NL // jblk,),
        in_specs=[
            pl.BlockSpec((_B, _D), lambda j: (0, 0)),
            pl.BlockSpec((jblk, _D), lambda j: (j, 0)),
            pl.BlockSpec((1, jblk), lambda j: (0, j)),
        ],
        out_specs=pl.BlockSpec((_B, jblk), lambda j: (0, j)),
        out_shape=jax.ShapeDtypeStruct((_B, _NL), jnp.float32),
    )(x, W_enc, b_enc.reshape(1, _NL))


_SC_MESH = plsc.VectorSubcoreMesh(core_axis_name="c", subcore_axis_name="s")
_SC_PARAMS = pltpu.CompilerParams(needs_layout_passes=False)

_UN = 16


def _zero_hist(hist_v, nb):
    def zbody(i, _):
        off = i * 256
        for u in range(16):
            hist_v[pl.ds(off + u * 16, 16)] = jnp.zeros((16,), jnp.int32)
        return 0
    lax.fori_loop(0, nb // 256, zbody, 0)


@functools.partial(
    pl.kernel,
    out_type=jax.ShapeDtypeStruct((_NW, _NB1), jnp.int32),
    mesh=_SC_MESH,
    compiler_params=_SC_PARAMS,
    scratch_types=[
        pltpu.VMEM((_NB1,), jnp.int32),
        pltpu.VMEM((_CH1,), jnp.float32),
        pltpu.VMEM((_CH1,), jnp.float32),
        pltpu.SemaphoreType.DMA,
        pltpu.SemaphoreType.DMA,
    ],
)
def _hist1(pre_hbm, out_hbm, hist_v, chunk0_v, chunk1_v, sem0, sem1):
    wid = lax.axis_index("s") * _NC + lax.axis_index("c")
    base = wid * _PER_W
    ones = jnp.ones((16,), jnp.int32)

    bufs = (chunk0_v, chunk1_v)
    for b in range(2):
        pltpu.async_copy(pre_hbm.at[pl.ds(base + b * _CH1, _CH1)],
                         bufs[b], (sem0, sem1)[b])
    _zero_hist(hist_v, _NB1)

    def cbody(k, _):
        for b in range(2):
            c = k * 2 + b
            sem = (sem0, sem1)[b]
            pltpu.make_async_copy(pre_hbm.at[pl.ds(base + c * _CH1, _CH1)],
                                  bufs[b], sem).wait()
            cv = bufs[b]

            @plsc.parallel_loop(0, _CH1 // 16, 1, unroll=_UN)
            def _(i):
                v = cv[pl.ds(i * 16, 16)]
                bits = plsc.bitcast(v, jnp.int32)
                plsc.addupdate_scatter(hist_v, [bits >> 16], ones,
                                       mask=bits != 0)

            nxt = c + 2

            @pl.when(nxt < (_PER_W // _CH1))
            def _():
                pltpu.async_copy(pre_hbm.at[pl.ds(base + nxt * _CH1, _CH1)],
                                 bufs[b], sem)
        return 0
    lax.fori_loop(0, (_PER_W // _CH1) // 2, cbody, 0)

    pltpu.sync_copy(hist_v, out_hbm.at[wid])


@functools.partial(
    pl.kernel,
    out_type=jax.ShapeDtypeStruct((_NW, _NB2), jnp.int32),
    mesh=_SC_MESH,
    compiler_params=_SC_PARAMS,
    scratch_types=[
        pltpu.VMEM((_NB2,), jnp.int32),
        pltpu.VMEM((_CH2,), jnp.float32),
        pltpu.VMEM((_CH2,), jnp.float32),
        pltpu.VMEM((16,), jnp.int32),
        pltpu.SemaphoreType.DMA,
        pltpu.SemaphoreType.DMA,
    ],
)
def _hist2(pre_hbm, b1_hbm, out_hbm, hist_v, chunk0_v, chunk1_v, bvec_v,
           sem0, sem1):
    wid = lax.axis_index("s") * _NC + lax.axis_index("c")
    base = wid * _PER_W
    ones = jnp.ones((16,), jnp.int32)

    bufs = (chunk0_v, chunk1_v)
    for b in range(2):
        pltpu.async_copy(pre_hbm.at[pl.ds(base + b * _CH2, _CH2)],
                         bufs[b], (sem0, sem1)[b])
    _zero_hist(hist_v, _NB2)
    pltpu.sync_copy(b1_hbm.at[pl.ds(0, 16)], bvec_v)
    b1 = bvec_v[pl.ds(0, 16)][0]

    def cbody(k, _):
        for b in range(2):
            c = k * 2 + b
            sem = (sem0, sem1)[b]
            pltpu.make_async_copy(pre_hbm.at[pl.ds(base + c * _CH2, _CH2)],
                                  bufs[b], sem).wait()
            cv = bufs[b]

            @plsc.parallel_loop(0, _CH2 // 16, 1, unroll=_UN)
            def _(i):
                v = cv[pl.ds(i * 16, 16)]
                bits = plsc.bitcast(v, jnp.int32)
                sel = (bits >> 16) == b1
                plsc.addupdate_scatter(hist_v, [bits & 0xFFFF], ones,
                                       mask=sel)

            nxt = c + 2

            @pl.when(nxt < (_PER_W // _CH2))
            def _():
                pltpu.async_copy(pre_hbm.at[pl.ds(base + nxt * _CH2, _CH2)],
                                 bufs[b], sem)
        return 0
    lax.fori_loop(0, (_PER_W // _CH2) // 2, cbody, 0)

    pltpu.sync_copy(hist_v, out_hbm.at[wid])


def _rank_search(hh, kf, nr):
    h2 = jnp.sum(hh.astype(jnp.float32), axis=0)
    ic = lax.broadcasted_iota(jnp.int32, (128, 128), 0)
    jc = lax.broadcasted_iota(jnp.int32, (128, 128), 1)
    tc = (ic >= jc).astype(jnp.float32)
    s_in = lax.dot_general(h2, tc, (((1,), (0,)), ((), ())),
                           preferred_element_type=jnp.float32,
                           precision=lax.Precision.HIGHEST)
    t = s_in[:, 0:1]
    ir = lax.broadcasted_iota(jnp.int32, (nr, nr), 0)
    jr = lax.broadcasted_iota(jnp.int32, (nr, nr), 1)
    ar = (jr > ir).astype(jnp.float32)
    rows_after = lax.dot_general(ar, t, (((1,), (0,)), ((), ())),
                                 preferred_element_type=jnp.float32,
                                 precision=lax.Precision.HIGHEST)
    s = s_in + rows_after

    cnt = jnp.sum((s >= kf).astype(jnp.float32))
    bstar = jnp.maximum(cnt.astype(jnp.int32) - 1, 0)
    one_hot = ((lax.broadcasted_iota(jnp.int32, (nr, 128), 0) == 0) &
               (lax.broadcasted_iota(jnp.int32, (nr, 128), 1) == 0))
    h00 = jnp.sum(jnp.where(one_hot, h2, 0.0))
    s00 = jnp.sum(jnp.where(one_hot, s, 0.0))
    s_next = jnp.where(cnt > 0.0,
                       jnp.max(jnp.where(s < kf, s, 0.0)),
                       s00 - h00)
    r = (kf - s_next).astype(jnp.int32)
    return bstar, r


def _finder_body(nr, kt_ref, h_ref, b_ref, r_ref):
    kf = kt_ref[0, 0].astype(jnp.float32)
    bstar, r = _rank_search(h_ref[...], kf, nr)
    b_ref[...] = jnp.full((8, 128), bstar, jnp.int32)
    r_ref[...] = jnp.full((8, 128), r, jnp.int32)


def _find_boundary(hists, ktarget, nb):
    nr = nb // 128
    return pl.pallas_call(
        functools.partial(_finder_body, nr),
        in_specs=[
            pl.BlockSpec(memory_space=pltpu.SMEM),
            pl.BlockSpec((_NW, nr, 128), lambda: (0, 0, 0)),
        ],
        out_specs=[
            pl.BlockSpec((8, 128), lambda: (0, 0)),
            pl.BlockSpec((8, 128), lambda: (0, 0)),
        ],
        out_shape=[
            jax.ShapeDtypeStruct((8, 128), jnp.int32),
            jax.ShapeDtypeStruct((8, 128), jnp.int32),
        ],
    )(ktarget, hists.reshape(_NW, nr, 128))


def _dec_body(thr_ref, pre_ref, w_ref, bd_ref, sp_ref, rec_ref):
    k = pl.program_id(0)
    t = thr_ref[0, 0]
    p = pre_ref[...]
    sp = jnp.where(p >= t, p, 0.0)
    sp_ref[...] = sp
    contrib = lax.dot_general(
        sp, w_ref[...], (((1,), (1,)), ((), ())),
        preferred_element_type=jnp.float32)

    @pl.when(k == 0)
    def _():
        rec_ref[...] = bd_ref[...] + contrib

    @pl.when(k > 0)
    def _():
        rec_ref[...] += contrib


def _decode(pre, W_dec, b_dec, thr):
    kblk = 512
    return pl.pallas_call(
        _dec_body,
        grid=(_NL // kblk,),
        in_specs=[
            pl.BlockSpec(memory_space=pltpu.SMEM),
            pl.BlockSpec((_B, kblk), lambda k: (0, k)),
            pl.BlockSpec((_D, kblk), lambda k: (0, k)),
            pl.BlockSpec((1, _D), lambda k: (0, 0)),
        ],
        out_specs=[
            pl.BlockSpec((_B, kblk), lambda k: (0, k)),
            pl.BlockSpec((_B, _D), lambda k: (0, 0)),
        ],
        out_shape=[
            jax.ShapeDtypeStruct((_B, _NL), jnp.float32),
            jax.ShapeDtypeStruct((_B, _D), jnp.float32),
        ],
    )(thr, pre, W_dec, b_dec.reshape(1, _D))


def kernel(x, W_enc, b_enc, W_dec, b_dec):
    pre = _encode(x, W_enc, b_enc)
    flat = pre.reshape(-1)
    h1 = _hist1(flat)
    kt = jnp.full((1, 1), _K_TOTAL, jnp.int32)
    b1_arr, r_arr = _find_boundary(h1, kt, _NB1)
    h2 = _hist2(flat, b1_arr.reshape(-1))
    b2_arr, _ = _find_boundary(h2, r_arr[0:1, 0:1], _NB2)
    tbits = (b1_arr[0, 0] << 16) | b2_arr[0, 0]
    thr = lax.bitcast_convert_type(tbits, jnp.float32).reshape(1, 1)
    sparse_acts, reconstruction = _decode(pre, W_dec, b_dec, thr)
    return (reconstruction, sparse_acts)


# speedup vs baseline: 69.8333x; 1.1489x over previous
import functools

import jax
import jax.numpy as jnp
from jax import lax
from jax.experimental import pallas as pl
from jax.experimental.pallas import tpu as pltpu
from jax.experimental.pallas import tpu_sc as plsc

_B = 2048
_D = 2048
_NL = 8192
_K_TOTAL = 32 * _B
_TOTAL = _B * _NL

_NW = 32
_NC = 2
_PER_W = _TOTAL // _NW
_CH1 = 32768
_CH2 = 16384
_NB1 = 1 << 15
_NB2 = 1 << 16


def _enc_body(x_ref, w_ref, b_ref, o_ref):
    acc = lax.dot_general(
        x_ref[...], w_ref[...], (((1,), (1,)), ((), ())),
        preferred_element_type=jnp.float32)
    o_ref[...] = jnp.maximum(acc + b_ref[...], 0.0)


def _encode(x, W_enc, b_enc):
    jblk = 1024
    return pl.pallas_call(
        _enc_body,
        grid=(_NL // jblk,),
        in_specs=[
            pl.BlockSpec((_B, _D), lambda j: (0, 0)),
            pl.BlockSpec((jblk, _D), lambda j: (j, 0)),
            pl.BlockSpec((1, jblk), lambda j: (0, j)),
        ],
        out_specs=pl.BlockSpec((_B, jblk), lambda j: (0, j)),
        out_shape=jax.ShapeDtypeStruct((_B, _NL), jnp.float32),
    )(x, W_enc, b_enc.reshape(1, _NL))


_SC_MESH = plsc.VectorSubcoreMesh(core_axis_name="c", subcore_axis_name="s")
_SC_PARAMS = pltpu.CompilerParams(needs_layout_passes=False)

_UN = 16


def _zero_hist(hist_v, nb):
    def zbody(i, _):
        off = i * 256
        for u in range(16):
            hist_v[pl.ds(off + u * 16, 16)] = jnp.zeros((16,), jnp.int32)
        return 0
    lax.fori_loop(0, nb // 256, zbody, 0)


_R1 = _CH1 // _NL
_R2 = _CH2 // _NL
_ROWS_W = _B // _NW


@functools.partial(
    pl.kernel,
    out_type=jax.ShapeDtypeStruct((_NW, _NB1), jnp.int32),
    mesh=_SC_MESH,
    compiler_params=_SC_PARAMS,
    scratch_types=[
        pltpu.VMEM((_NB1,), jnp.int32),
        pltpu.VMEM((_R1, _NL), jnp.float32),
        pltpu.VMEM((_R1, _NL), jnp.float32),
        pltpu.SemaphoreType.DMA,
        pltpu.SemaphoreType.DMA,
    ],
)
def _hist1(pre_hbm, out_hbm, hist_v, chunk0_v, chunk1_v, sem0, sem1):
    wid = lax.axis_index("s") * _NC + lax.axis_index("c")
    base = wid * _ROWS_W
    ones = jnp.ones((16,), jnp.int32)
    nch = _ROWS_W // _R1

    bufs = (chunk0_v, chunk1_v)
    for b in range(2):
        pltpu.async_copy(pre_hbm.at[pl.ds(base + b * _R1, _R1)],
                         bufs[b], (sem0, sem1)[b])
    _zero_hist(hist_v, _NB1)

    def cbody(k, _):
        for b in range(2):
            c = k * 2 + b
            sem = (sem0, sem1)[b]
            pltpu.make_async_copy(pre_hbm.at[pl.ds(base + c * _R1, _R1)],
                                  bufs[b], sem).wait()
            cv = bufs[b]

            for r in range(_R1):
                @plsc.parallel_loop(0, _NL // 16, 1, unroll=_UN)
                def _(i):
                    v = cv[r, pl.ds(i * 16, 16)]
                    bits = plsc.bitcast(v, jnp.int32)
                    plsc.addupdate_scatter(hist_v, [bits >> 16], ones,
                                           mask=bits != 0)

            nxt = c + 2

            @pl.when(nxt < nch)
            def _():
                pltpu.async_copy(pre_hbm.at[pl.ds(base + nxt * _R1, _R1)],
                                 bufs[b], sem)
        return 0
    lax.fori_loop(0, nch // 2, cbody, 0)

    pltpu.sync_copy(hist_v, out_hbm.at[wid])


@functools.partial(
    pl.kernel,
    out_type=jax.ShapeDtypeStruct((_NW, _NB2), jnp.int32),
    mesh=_SC_MESH,
    compiler_params=_SC_PARAMS,
    scratch_types=[
        pltpu.VMEM((_NB2,), jnp.int32),
        pltpu.VMEM((_R2, _NL), jnp.float32),
        pltpu.VMEM((_R2, _NL), jnp.float32),
        pltpu.VMEM((16,), jnp.int32),
        pltpu.SemaphoreType.DMA,
        pltpu.SemaphoreType.DMA,
    ],
)
def _hist2(pre_hbm, b1_hbm, out_hbm, hist_v, chunk0_v, chunk1_v, bvec_v,
           sem0, sem1):
    wid = lax.axis_index("s") * _NC + lax.axis_index("c")
    base = wid * _ROWS_W
    ones = jnp.ones((16,), jnp.int32)
    nch = _ROWS_W // _R2

    bufs = (chunk0_v, chunk1_v)
    for b in range(2):
        pltpu.async_copy(pre_hbm.at[pl.ds(base + b * _R2, _R2)],
                         bufs[b], (sem0, sem1)[b])
    _zero_hist(hist_v, _NB2)
    pltpu.sync_copy(b1_hbm.at[pl.ds(0, 16)], bvec_v)
    b1 = bvec_v[pl.ds(0, 16)][0]

    def cbody(k, _):
        for b in range(2):
            c = k * 2 + b
            sem = (sem0, sem1)[b]
            pltpu.make_async_copy(pre_hbm.at[pl.ds(base + c * _R2, _R2)],
                                  bufs[b], sem).wait()
            cv = bufs[b]

            for r in range(_R2):
                @plsc.parallel_loop(0, _NL // 16, 1, unroll=_UN)
                def _(i):
                    v = cv[r, pl.ds(i * 16, 16)]
                    bits = plsc.bitcast(v, jnp.int32)
                    sel = (bits >> 16) == b1
                    plsc.addupdate_scatter(hist_v, [bits & 0xFFFF], ones,
                                           mask=sel)

            nxt = c + 2

            @pl.when(nxt < nch)
            def _():
                pltpu.async_copy(pre_hbm.at[pl.ds(base + nxt * _R2, _R2)],
                                 bufs[b], sem)
        return 0
    lax.fori_loop(0, nch // 2, cbody, 0)

    pltpu.sync_copy(hist_v, out_hbm.at[wid])


def _rank_search(hh, kf, nr):
    h2 = jnp.sum(hh.astype(jnp.float32), axis=0)
    ic = lax.broadcasted_iota(jnp.int32, (128, 128), 0)
    jc = lax.broadcasted_iota(jnp.int32, (128, 128), 1)
    tc = (ic >= jc).astype(jnp.float32)
    s_in = lax.dot_general(h2, tc, (((1,), (0,)), ((), ())),
                           preferred_element_type=jnp.float32,
                           precision=lax.Precision.HIGHEST)
    t = s_in[:, 0:1]
    ir = lax.broadcasted_iota(jnp.int32, (nr, nr), 0)
    jr = lax.broadcasted_iota(jnp.int32, (nr, nr), 1)
    ar = (jr > ir).astype(jnp.float32)
    rows_after = lax.dot_general(ar, t, (((1,), (0,)), ((), ())),
                                 preferred_element_type=jnp.float32,
                                 precision=lax.Precision.HIGHEST)
    s = s_in + rows_after

    cnt = jnp.sum((s >= kf).astype(jnp.float32))
    bstar = jnp.maximum(cnt.astype(jnp.int32) - 1, 0)
    one_hot = ((lax.broadcasted_iota(jnp.int32, (nr, 128), 0) == 0) &
               (lax.broadcasted_iota(jnp.int32, (nr, 128), 1) == 0))
    h00 = jnp.sum(jnp.where(one_hot, h2, 0.0))
    s00 = jnp.sum(jnp.where(one_hot, s, 0.0))
    s_next = jnp.where(cnt > 0.0,
                       jnp.max(jnp.where(s < kf, s, 0.0)),
                       s00 - h00)
    r = (kf - s_next).astype(jnp.int32)
    return bstar, r


def _finder_body(nr, kt_ref, h_ref, b_ref, r_ref):
    kf = kt_ref[0, 0].astype(jnp.float32)
    bstar, r = _rank_search(h_ref[...], kf, nr)
    b_ref[...] = jnp.full((8, 128), bstar, jnp.int32)
    r_ref[...] = jnp.full((8, 128), r, jnp.int32)


def _find_boundary(hists, ktarget, nb):
    nr = nb // 128
    return pl.pallas_call(
        functools.partial(_finder_body, nr),
        in_specs=[
            pl.BlockSpec(memory_space=pltpu.SMEM),
            pl.BlockSpec((_NW, nr, 128), lambda: (0, 0, 0)),
        ],
        out_specs=[
            pl.BlockSpec((8, 128), lambda: (0, 0)),
            pl.BlockSpec((8, 128), lambda: (0, 0)),
        ],
        out_shape=[
            jax.ShapeDtypeStruct((8, 128), jnp.int32),
            jax.ShapeDtypeStruct((8, 128), jnp.int32),
        ],
    )(ktarget, hists.reshape(_NW, nr, 128))


def _dec_body(thr_ref, pre_ref, w_ref, bd_ref, sp_ref, rec_ref):
    k = pl.program_id(0)
    t = thr_ref[0, 0]
    p = pre_ref[...]
    sp = jnp.where(p >= t, p, 0.0)
    sp_ref[...] = sp
    contrib = lax.dot_general(
        sp, w_ref[...], (((1,), (1,)), ((), ())),
        preferred_element_type=jnp.float32)

    @pl.when(k == 0)
    def _():
        rec_ref[...] = bd_ref[...] + contrib

    @pl.when(k > 0)
    def _():
        rec_ref[...] += contrib


def _decode(pre, W_dec, b_dec, thr):
    kblk = 512
    return pl.pallas_call(
        _dec_body,
        grid=(_NL // kblk,),
        in_specs=[
            pl.BlockSpec(memory_space=pltpu.SMEM),
            pl.BlockSpec((_B, kblk), lambda k: (0, k)),
            pl.BlockSpec((_D, kblk), lambda k: (0, k)),
            pl.BlockSpec((1, _D), lambda k: (0, 0)),
        ],
        out_specs=[
            pl.BlockSpec((_B, kblk), lambda k: (0, k)),
            pl.BlockSpec((_B, _D), lambda k: (0, 0)),
        ],
        out_shape=[
            jax.ShapeDtypeStruct((_B, _NL), jnp.float32),
            jax.ShapeDtypeStruct((_B, _D), jnp.float32),
        ],
    )(thr, pre, W_dec, b_dec.reshape(1, _D))


def kernel(x, W_enc, b_enc, W_dec, b_dec):
    pre = _encode(x, W_enc, b_enc)
    h1 = _hist1(pre)
    kt = jnp.full((1, 1), _K_TOTAL, jnp.int32)
    b1_arr, r_arr = _find_boundary(h1, kt, _NB1)
    h2 = _hist2(pre, b1_arr.reshape(-1))
    b2_arr, _ = _find_boundary(h2, r_arr[0:1, 0:1], _NB2)
    tbits = (b1_arr[0, 0] << 16) | b2_arr[0, 0]
    thr = lax.bitcast_convert_type(tbits, jnp.float32).reshape(1, 1)
    sparse_acts, reconstruction = _decode(pre, W_dec, b_dec, thr)
    return (reconstruction, sparse_acts)
